# Initial kernel scaffold; baseline (speedup 1.0000x reference)
import jax
import jax.numpy as jnp
from jax.experimental import pallas as pl


def kernel(nodes, graph, cluster, W1, b1, W2, b2, W3, b3, b_gamma, b_beta, s_gamma, s_beta):
    raise NotImplementedError("write your pallas kernel here")


# speedup vs baseline: 5.2594x; 5.2594x over previous
import functools

import jax
import jax.numpy as jnp
from jax.experimental import pallas as pl
from jax.experimental.pallas import tpu as pltpu

N = 10000
NC = 1000
D_MODEL = 128
D_HIDDEN = 256
EMB = 16
BK = 5
SK = 10
E = 320000

_ROWS = 1000
_NBLK = N // _ROWS

_NEG_BIG = -1.0e30
_IDX_BIG = 2**30


def _mlp_body(x_ref, w1_ref, b1_ref, w2_ref, b2_ref, w3_ref, b3_ref, o_ref):
    x = x_ref[...]
    h = jnp.dot(x, w1_ref[...], preferred_element_type=jnp.float32) + b1_ref[...]
    h = jax.nn.gelu(h)
    h = jnp.dot(h, w2_ref[...], preferred_element_type=jnp.float32) + b2_ref[...]
    h = jax.nn.gelu(h)
    o_ref[...] = jnp.dot(h, w3_ref[...], preferred_element_type=jnp.float32) + b3_ref[...]


def _mlp(nodes, W1, b1, W2, b2, W3, b3):
    full = lambda shape: pl.BlockSpec(shape, lambda i: (0,) * len(shape))
    return pl.pallas_call(
        _mlp_body,
        grid=(_NBLK,),
        in_specs=[
            pl.BlockSpec((_ROWS, D_MODEL), lambda i: (i, 0)),
            full((D_MODEL, D_HIDDEN)),
            full((1, D_HIDDEN)),
            full((D_HIDDEN, D_HIDDEN)),
            full((1, D_HIDDEN)),
            full((D_HIDDEN, EMB)),
            full((1, EMB)),
        ],
        out_specs=pl.BlockSpec((_ROWS, EMB), lambda i: (i, 0)),
        out_shape=jax.ShapeDtypeStruct((N, EMB), jnp.float32),
    )(nodes, W1, b1.reshape(1, -1), W2, b2.reshape(1, -1), W3, b3.reshape(1, -1))


def _pool_body(emb_ref, c0_ref, c1_ref, sums_ref, cnt_ref):
    i = pl.program_id(0)

    @pl.when(i == 0)
    def _init():
        sums_ref[...] = jnp.zeros_like(sums_ref)
        cnt_ref[...] = jnp.zeros_like(cnt_ref)

    emb1k = emb_ref[...]
    c0 = c0_ref[...]
    c1 = c1_ref[...]
    iota_c = jax.lax.broadcasted_iota(jnp.int32, (_ROWS, NC), 1)
    oh_src = (c0 == iota_c).astype(jnp.float32)
    oh_dst = (c1 == iota_c).astype(jnp.float32)
    g = jax.lax.dot_general(oh_src, emb1k, (((1,), (0,)), ((), ())),
                            preferred_element_type=jnp.float32,
                            precision=jax.lax.Precision.HIGHEST)
    sums_ref[...] += jax.lax.dot_general(oh_dst, g, (((0,), (0,)), ((), ())),
                                         preferred_element_type=jnp.float32,
                                         precision=jax.lax.Precision.HIGHEST)
    ones = jnp.ones((_ROWS, 1), jnp.float32)
    cnt_ref[...] += jax.lax.dot_general(oh_dst, ones, (((0,), (0,)), ((), ())),
                                        preferred_element_type=jnp.float32,
                                        precision=jax.lax.Precision.HIGHEST)


def _pool(emb, c0, c1):
    return pl.pallas_call(
        _pool_body,
        grid=(_NBLK,),
        in_specs=[
            pl.BlockSpec((NC, EMB), lambda i: (0, 0)),
            pl.BlockSpec((_ROWS, 1), lambda i: (i, 0)),
            pl.BlockSpec((_ROWS, 1), lambda i: (i, 0)),
        ],
        out_specs=(
            pl.BlockSpec((NC, EMB), lambda i: (0, 0)),
            pl.BlockSpec((NC, 1), lambda i: (0, 0)),
        ),
        out_shape=(
            jax.ShapeDtypeStruct((NC, EMB), jnp.float32),
            jax.ShapeDtypeStruct((NC, 1), jnp.float32),
        ),
    )(emb, c0, c1)


def _topk_iter(neg, k):
    iota = jax.lax.broadcasted_iota(jnp.int32, neg.shape, 1)
    vals, idxs = [], []
    cur = neg
    for _ in range(k):
        m = jnp.max(cur, axis=1, keepdims=True)
        idx = jnp.min(jnp.where(cur == m, iota, _IDX_BIG), axis=1, keepdims=True)
        vals.append(m)
        idxs.append(idx)
        cur = jnp.where(iota == idx, _NEG_BIG, cur)
    return jnp.concatenate(vals, axis=1), jnp.concatenate(idxs, axis=1)


def _neg_sqdist(a, b):
    asq = jnp.sum(a * a, axis=1, keepdims=True)
    bsq_row = jax.lax.dot_general(jnp.ones((1, b.shape[1]), jnp.float32), b * b,
                                  (((1,), (1,)), ((), ())),
                                  preferred_element_type=jnp.float32,
                                  precision=jax.lax.Precision.HIGHEST)
    ab = jax.lax.dot_general(a, b, (((1,), (1,)), ((), ())),
                             preferred_element_type=jnp.float32)
    d2 = (asq - 2.0 * ab) + bsq_row
    return -d2


def _bip_body(emb_ref, sums_ref, cnt_ref, bg_ref, bb_ref,
              idx_ref, logit_ref, w_ref):
    emb = emb_ref[...]
    cnt = cnt_ref[...]
    semb = sums_ref[...] * (1.0 / jnp.clip(cnt, 1.0, None))
    neg = _neg_sqdist(emb, semb)
    vals, idxs = _topk_iter(neg, BK)
    bg = bg_ref[0, 0]
    bb = bb_ref[0, 0]
    logits = bg * vals + bb
    bw = jnp.exp(logits)
    den = jnp.sum(bw, axis=1, keepdims=True)
    idx_ref[...] = idxs
    logit_ref[...] = logits
    w_ref[...] = bw / (1e-12 + den)


def _bip(emb, sums, cnt, bg, bb):
    return pl.pallas_call(
        _bip_body,
        grid=(_NBLK,),
        in_specs=[
            pl.BlockSpec((_ROWS, EMB), lambda i: (i, 0)),
            pl.BlockSpec((NC, EMB), lambda i: (0, 0)),
            pl.BlockSpec((NC, 1), lambda i: (0, 0)),
            pl.BlockSpec((1, 1), lambda i: (0, 0)),
            pl.BlockSpec((1, 1), lambda i: (0, 0)),
        ],
        out_specs=(
            pl.BlockSpec((_ROWS, BK), lambda i: (i, 0)),
            pl.BlockSpec((_ROWS, BK), lambda i: (i, 0)),
            pl.BlockSpec((_ROWS, BK), lambda i: (i, 0)),
        ),
        out_shape=(
            jax.ShapeDtypeStruct((N, BK), jnp.int32),
            jax.ShapeDtypeStruct((N, BK), jnp.float32),
            jax.ShapeDtypeStruct((N, BK), jnp.float32),
        ),
    )(emb, sums, cnt, bg.reshape(1, 1), bb.reshape(1, 1))


def _snn_body(sums_ref, cnt_ref, sg_ref, sb_ref,
              semb_ref, idx_ref, w1_ref, w2_ref):
    cnt = cnt_ref[...]
    semb = sums_ref[...] * (1.0 / jnp.clip(cnt, 1.0, None))
    semb_ref[...] = semb
    neg = _neg_sqdist(semb, semb)
    vals, idxs = _topk_iter(neg, SK)
    sg = sg_ref[0, 0]
    sb = sb_ref[0, 0]
    sw = jax.nn.sigmoid(sg * vals + sb)
    iota_r = jax.lax.broadcasted_iota(jnp.int32, (NC, NC), 0)
    iota_c = jax.lax.broadcasted_iota(jnp.int32, (NC, NC), 1)
    diag = iota_r == iota_c
    out_col = jnp.sum(sw, axis=1, keepdims=True)
    s_acc = jnp.zeros((NC, NC), jnp.float32)
    for k in range(SK):
        s_acc += jnp.where(idxs[:, k:k + 1] == iota_c, sw[:, k:k + 1], 0.0)
    in_row = jnp.sum(s_acc, axis=0, keepdims=True)
    out_row = jnp.sum(jnp.where(diag, out_col, 0.0), axis=0, keepdims=True)
    sden_row = out_row + in_row
    sden_col = jnp.sum(jnp.where(diag, sden_row, 0.0), axis=1, keepdims=True)
    w1_ref[...] = sw / (1e-12 + sden_col)
    cols = []
    for k in range(SK):
        picked = jnp.where(idxs[:, k:k + 1] == iota_c, sden_row, 0.0)
        sden_at = jnp.sum(picked, axis=1, keepdims=True)
        cols.append(sw[:, k:k + 1] / (1e-12 + sden_at))
    w2_ref[...] = jnp.concatenate(cols, axis=1)
    idx_ref[...] = idxs


def _snn(sums, cnt, sg, sb):
    full = lambda shape: pl.BlockSpec(shape, lambda: (0,) * len(shape))
    return pl.pallas_call(
        _snn_body,
        in_specs=[
            full((NC, EMB)),
            full((NC, 1)),
            full((1, 1)),
            full((1, 1)),
        ],
        out_specs=(
            full((NC, EMB)),
            full((NC, SK)),
            full((NC, SK)),
            full((NC, SK)),
        ),
        out_shape=(
            jax.ShapeDtypeStruct((NC, EMB), jnp.float32),
            jax.ShapeDtypeStruct((NC, SK), jnp.int32),
            jax.ShapeDtypeStruct((NC, SK), jnp.float32),
            jax.ShapeDtypeStruct((NC, SK), jnp.float32),
        ),
    )(sums, cnt, sg.reshape(1, 1), sb.reshape(1, 1))


def kernel(nodes, graph, cluster, W1, b1, W2, b2, W3, b3,
           b_gamma, b_beta, s_gamma, s_beta):
    emb = _mlp(nodes, W1, b1, W2, b2, W3, b3)
    c0 = cluster[0].reshape(N, 1)
    c1 = cluster[1].reshape(N, 1)
    sums, cnt = _pool(emb[:NC], c0, c1)
    bnn, blogits5, bw5 = _bip(emb, sums, cnt, b_gamma, b_beta)
    semb, snn, sw1, sw2 = _snn(sums, cnt, s_gamma, s_beta)

    bsrc = jnp.repeat(jnp.arange(N, dtype=jnp.int32), BK)
    bgraph = jnp.stack([bsrc, bnn.reshape(-1)], axis=0)
    bweights = bw5.reshape(-1, 1)
    blogits = blogits5.reshape(-1)

    s0 = jnp.repeat(jnp.arange(NC, dtype=jnp.int32), SK)
    s1 = snn.reshape(-1)
    sgraph = jnp.stack([jnp.concatenate([s0, s1]),
                        jnp.concatenate([s1, s0])], axis=0)
    sweights = jnp.concatenate([sw1.reshape(-1), sw2.reshape(-1)])[:, None]

    mask = jnp.ones((E,), dtype=bool)
    return (emb, semb, bgraph, bweights, sgraph, sweights, blogits, mask)


# speedup vs baseline: 9.4393x; 1.7948x over previous
import functools

import jax
import jax.numpy as jnp
from jax import lax
from jax.experimental import pallas as pl
from jax.experimental.pallas import tpu as pltpu
from jax.experimental.pallas import tpu_sc as plsc

N = 10000
NC = 1000
D_MODEL = 128
D_HIDDEN = 256
EMB = 16
BK = 5
SK = 10
E = 320000

_ROWS = 1000
_NBLK = N // _ROWS

_NEG_BIG = -1.0e30
_IDX_BIG = 2**30


def _mlp_body(x_ref, w1_ref, b1_ref, w2_ref, b2_ref, w3_ref, b3_ref, o_ref):
    x = x_ref[...]
    h = jnp.dot(x, w1_ref[...], preferred_element_type=jnp.float32) + b1_ref[...]
    h = jax.nn.gelu(h)
    h = jnp.dot(h, w2_ref[...], preferred_element_type=jnp.float32) + b2_ref[...]
    h = jax.nn.gelu(h)
    o_ref[...] = jnp.dot(h, w3_ref[...], preferred_element_type=jnp.float32) + b3_ref[...]


def _mlp(nodes, W1, b1, W2, b2, W3, b3):
    full = lambda shape: pl.BlockSpec(shape, lambda i: (0,) * len(shape))
    return pl.pallas_call(
        _mlp_body,
        grid=(_NBLK,),
        in_specs=[
            pl.BlockSpec((_ROWS, D_MODEL), lambda i: (i, 0)),
            full((D_MODEL, D_HIDDEN)),
            full((1, D_HIDDEN)),
            full((D_HIDDEN, D_HIDDEN)),
            full((1, D_HIDDEN)),
            full((D_HIDDEN, EMB)),
            full((1, EMB)),
        ],
        out_specs=pl.BlockSpec((_ROWS, EMB), lambda i: (i, 0)),
        out_shape=jax.ShapeDtypeStruct((N, EMB), jnp.float32),
    )(nodes, W1, b1.reshape(1, -1), W2, b2.reshape(1, -1), W3, b3.reshape(1, -1))


_NW = 32
_PW = 320
_CH = 4
_CW = 80
_RA = 1024
_TR = _RA // 16


_WIDE = 128


def _pool_sc_body(embw, c0p, c1p, acc_out,
                  idx0_v, idx1_v, rows_v, zero_v, shared_a, sem):
    cid = lax.axis_index("c")
    sid = lax.axis_index("s")
    wid = sid * 2 + cid

    pltpu.sync_copy(c0p.at[wid], idx0_v)
    pltpu.sync_copy(c1p.at[wid], idx1_v)

    zvec = jnp.zeros((16,), jnp.float32)

    def _fill_zero(r, carry):
        for q in range(_WIDE // 16):
            zero_v[r, pl.ds(q * 16, 16)] = zvec
        return carry

    lax.fori_loop(0, _TR, _fill_zero, 0)

    row0 = sid * _TR
    pltpu.sync_copy(zero_v, shared_a.at[pl.ds(row0, _TR)])
    plsc.subcore_barrier()

    for j in range(_CH):
        pltpu.async_copy(embw.at[idx0_v.at[j]], rows_v.at[j], sem).wait()
    for j in range(_CH):
        pltpu.sync_copy(rows_v.at[j], shared_a.at[idx1_v.at[j]], add=True)
    plsc.subcore_barrier()

    out0 = cid * _RA + row0
    pltpu.sync_copy(shared_a.at[pl.ds(row0, _TR)], acc_out.at[pl.ds(out0, _TR)])


_pool_sc_call = functools.partial(
    pl.kernel,
    out_type=jax.ShapeDtypeStruct((2 * _RA, _WIDE), jnp.float32),
    mesh=plsc.VectorSubcoreMesh(core_axis_name="c", subcore_axis_name="s"),
    scratch_types=[
        pltpu.VMEM((_CH, _CW), jnp.int32),
        pltpu.VMEM((_CH, _CW), jnp.int32),
        pltpu.VMEM((_CH, _CW, _WIDE), jnp.float32),
        pltpu.VMEM((_TR, _WIDE), jnp.float32),
        pltpu.VMEM_SHARED((_RA, _WIDE), jnp.float32),
        pltpu.SemaphoreType.DMA,
    ],
)(_pool_sc_body)


def _pool(emb1k, c0, c1):
    embw = jnp.concatenate(
        [emb1k, jnp.ones((NC, 1), jnp.float32),
         jnp.zeros((NC, _WIDE - EMB - 1), jnp.float32)], axis=1)
    pad = _NW * _PW - N
    c0p = jnp.concatenate([c0, jnp.zeros((pad,), jnp.int32)]).reshape(_NW, _CH, _CW)
    c1p = jnp.concatenate([c1, jnp.full((pad,), NC, jnp.int32)]).reshape(_NW, _CH, _CW)
    return _pool_sc_call(embw, c0p, c1p)


def _topk_iter(neg, k):
    iota = jax.lax.broadcasted_iota(jnp.int32, neg.shape, 1)
    vals, idxs = [], []
    cur = neg
    for _ in range(k):
        m = jnp.max(cur, axis=1, keepdims=True)
        idx = jnp.min(jnp.where(cur == m, iota, _IDX_BIG), axis=1, keepdims=True)
        vals.append(m)
        idxs.append(idx)
        cur = jnp.where(iota == idx, _NEG_BIG, cur)
    return jnp.concatenate(vals, axis=1), jnp.concatenate(idxs, axis=1)


def _neg_sqdist(a, b):
    asq = jnp.sum(a * a, axis=1, keepdims=True)
    bsq_row = jax.lax.dot_general(jnp.ones((1, b.shape[1]), jnp.float32), b * b,
                                  (((1,), (1,)), ((), ())),
                                  preferred_element_type=jnp.float32,
                                  precision=jax.lax.Precision.HIGHEST)
    ab = jax.lax.dot_general(a, b, (((1,), (1,)), ((), ())),
                             preferred_element_type=jnp.float32)
    d2 = (asq - 2.0 * ab) + bsq_row
    return -d2


def _combine_partials(acc_ref):
    a = acc_ref[...]
    sums = a[0:NC, 0:EMB] + a[_RA:_RA + NC, 0:EMB]
    cnt = a[0:NC, EMB:EMB + 1] + a[_RA:_RA + NC, EMB:EMB + 1]
    return sums * (1.0 / jnp.clip(cnt, 1.0, None))


def _bip_body(emb_ref, acc_ref, bg_ref, bb_ref,
              idx_ref, logit_ref, w_ref):
    emb = emb_ref[...]
    semb = _combine_partials(acc_ref)
    neg = _neg_sqdist(emb, semb)
    vals, idxs = _topk_iter(neg, BK)
    bg = bg_ref[0, 0]
    bb = bb_ref[0, 0]
    logits = bg * vals + bb
    bw = jnp.exp(logits)
    den = jnp.sum(bw, axis=1, keepdims=True)
    idx_ref[...] = idxs
    logit_ref[...] = logits
    w_ref[...] = bw / (1e-12 + den)


def _bip(emb, acc, bg, bb):
    return pl.pallas_call(
        _bip_body,
        grid=(_NBLK,),
        in_specs=[
            pl.BlockSpec((_ROWS, EMB), lambda i: (i, 0)),
            pl.BlockSpec((2 * _RA, _WIDE), lambda i: (0, 0)),
            pl.BlockSpec((1, 1), lambda i: (0, 0)),
            pl.BlockSpec((1, 1), lambda i: (0, 0)),
        ],
        out_specs=(
            pl.BlockSpec((_ROWS, BK), lambda i: (i, 0)),
            pl.BlockSpec((_ROWS, BK), lambda i: (i, 0)),
            pl.BlockSpec((_ROWS, BK), lambda i: (i, 0)),
        ),
        out_shape=(
            jax.ShapeDtypeStruct((N, BK), jnp.int32),
            jax.ShapeDtypeStruct((N, BK), jnp.float32),
            jax.ShapeDtypeStruct((N, BK), jnp.float32),
        ),
    )(emb, acc, bg.reshape(1, 1), bb.reshape(1, 1))


def _snn_body(acc_ref, sg_ref, sb_ref,
              semb_ref, idx_ref, w1_ref, w2_ref):
    semb = _combine_partials(acc_ref)
    semb_ref[...] = semb
    neg = _neg_sqdist(semb, semb)
    vals, idxs = _topk_iter(neg, SK)
    sg = sg_ref[0, 0]
    sb = sb_ref[0, 0]
    sw = jax.nn.sigmoid(sg * vals + sb)
    iota_r = jax.lax.broadcasted_iota(jnp.int32, (NC, NC), 0)
    iota_c = jax.lax.broadcasted_iota(jnp.int32, (NC, NC), 1)
    diag = iota_r == iota_c
    out_col = jnp.sum(sw, axis=1, keepdims=True)
    s_acc = jnp.zeros((NC, NC), jnp.float32)
    for k in range(SK):
        s_acc += jnp.where(idxs[:, k:k + 1] == iota_c, sw[:, k:k + 1], 0.0)
    in_row = jnp.sum(s_acc, axis=0, keepdims=True)
    out_row = jnp.sum(jnp.where(diag, out_col, 0.0), axis=0, keepdims=True)
    sden_row = out_row + in_row
    sden_col = jnp.sum(jnp.where(diag, sden_row, 0.0), axis=1, keepdims=True)
    w1_ref[...] = sw / (1e-12 + sden_col)
    cols = []
    for k in range(SK):
        picked = jnp.where(idxs[:, k:k + 1] == iota_c, sden_row, 0.0)
        sden_at = jnp.sum(picked, axis=1, keepdims=True)
        cols.append(sw[:, k:k + 1] / (1e-12 + sden_at))
    w2_ref[...] = jnp.concatenate(cols, axis=1)
    idx_ref[...] = idxs


def _snn(acc, sg, sb):
    full = lambda shape: pl.BlockSpec(shape, lambda: (0,) * len(shape))
    return pl.pallas_call(
        _snn_body,
        in_specs=[
            full((2 * _RA, _WIDE)),
            full((1, 1)),
            full((1, 1)),
        ],
        out_specs=(
            full((NC, EMB)),
            full((NC, SK)),
            full((NC, SK)),
            full((NC, SK)),
        ),
        out_shape=(
            jax.ShapeDtypeStruct((NC, EMB), jnp.float32),
            jax.ShapeDtypeStruct((NC, SK), jnp.int32),
            jax.ShapeDtypeStruct((NC, SK), jnp.float32),
            jax.ShapeDtypeStruct((NC, SK), jnp.float32),
        ),
    )(acc, sg.reshape(1, 1), sb.reshape(1, 1))


def kernel(nodes, graph, cluster, W1, b1, W2, b2, W3, b3,
           b_gamma, b_beta, s_gamma, s_beta):
    emb = _mlp(nodes, W1, b1, W2, b2, W3, b3)
    acc = _pool(emb[:NC], cluster[0], cluster[1])
    bnn, blogits5, bw5 = _bip(emb, acc, b_gamma, b_beta)
    semb, snn, sw1, sw2 = _snn(acc, s_gamma, s_beta)

    bsrc = jnp.repeat(jnp.arange(N, dtype=jnp.int32), BK)
    bgraph = jnp.stack([bsrc, bnn.reshape(-1)], axis=0)
    bweights = bw5.reshape(-1, 1)
    blogits = blogits5.reshape(-1)

    s0 = jnp.repeat(jnp.arange(NC, dtype=jnp.int32), SK)
    s1 = snn.reshape(-1)
    sgraph = jnp.stack([jnp.concatenate([s0, s1]),
                        jnp.concatenate([s1, s0])], axis=0)
    sweights = jnp.concatenate([sw1.reshape(-1), sw2.reshape(-1)])[:, None]

    mask = jnp.ones((E,), dtype=bool)
    return (emb, semb, bgraph, bweights, sgraph, sweights, blogits, mask)


# speedup vs baseline: 10.3557x; 1.0971x over previous
import functools

import jax
import jax.numpy as jnp
from jax import lax
from jax.experimental import pallas as pl
from jax.experimental.pallas import tpu as pltpu
from jax.experimental.pallas import tpu_sc as plsc

N = 10000
NC = 1000
D_MODEL = 128
D_HIDDEN = 256
EMB = 16
BK = 5
SK = 10
E = 320000

_ROWS = 1000
_NBLK = N // _ROWS

_NEG_BIG = -1.0e30
_IDX_BIG = 2**30


def _mlp_body(x_ref, w1_ref, b1_ref, w2_ref, b2_ref, w3_ref, b3_ref, o_ref):
    x = x_ref[...]
    h = jnp.dot(x, w1_ref[...], preferred_element_type=jnp.float32) + b1_ref[...]
    h = jax.nn.gelu(h)
    h = jnp.dot(h, w2_ref[...], preferred_element_type=jnp.float32) + b2_ref[...]
    h = jax.nn.gelu(h)
    o_ref[...] = jnp.dot(h, w3_ref[...], preferred_element_type=jnp.float32) + b3_ref[...]


def _mlp(nodes, W1, b1, W2, b2, W3, b3):
    full = lambda shape: pl.BlockSpec(shape, lambda i: (0,) * len(shape))
    return pl.pallas_call(
        _mlp_body,
        grid=(_NBLK,),
        in_specs=[
            pl.BlockSpec((_ROWS, D_MODEL), lambda i: (i, 0)),
            full((D_MODEL, D_HIDDEN)),
            full((1, D_HIDDEN)),
            full((D_HIDDEN, D_HIDDEN)),
            full((1, D_HIDDEN)),
            full((D_HIDDEN, EMB)),
            full((1, EMB)),
        ],
        out_specs=pl.BlockSpec((_ROWS, EMB), lambda i: (i, 0)),
        out_shape=jax.ShapeDtypeStruct((N, EMB), jnp.float32),
    )(nodes, W1, b1.reshape(1, -1), W2, b2.reshape(1, -1), W3, b3.reshape(1, -1))


_NW = 32
_PW = 320
_CH = 4
_CW = 80
_RA = 1024
_TR = _RA // 16


_WIDE = 128


def _pool_sc_body(embw, c0p, c1p, acc_out,
                  idx0_v, idx1_v, rows_v, zero_v, shared_a, sem):
    cid = lax.axis_index("c")
    sid = lax.axis_index("s")
    wid = sid * 2 + cid

    pltpu.sync_copy(c0p.at[wid], idx0_v)
    pltpu.sync_copy(c1p.at[wid], idx1_v)

    zvec = jnp.zeros((16,), jnp.float32)

    def _fill_zero(r, carry):
        for q in range(_WIDE // 16):
            zero_v[r, pl.ds(q * 16, 16)] = zvec
        return carry

    lax.fori_loop(0, _TR, _fill_zero, 0)

    row0 = sid * _TR
    pltpu.sync_copy(zero_v, shared_a.at[pl.ds(row0, _TR)])
    plsc.subcore_barrier()

    for j in range(_CH):
        pltpu.async_copy(embw.at[idx0_v.at[j]], rows_v.at[j], sem).wait()
    for j in range(_CH):
        pltpu.sync_copy(rows_v.at[j], shared_a.at[idx1_v.at[j]], add=True)
    plsc.subcore_barrier()

    out0 = cid * _RA + row0
    pltpu.sync_copy(shared_a.at[pl.ds(row0, _TR)], acc_out.at[pl.ds(out0, _TR)])


_pool_sc_call = functools.partial(
    pl.kernel,
    out_type=jax.ShapeDtypeStruct((2 * _RA, _WIDE), jnp.float32),
    mesh=plsc.VectorSubcoreMesh(core_axis_name="c", subcore_axis_name="s"),
    scratch_types=[
        pltpu.VMEM((_CH, _CW), jnp.int32),
        pltpu.VMEM((_CH, _CW), jnp.int32),
        pltpu.VMEM((_CH, _CW, _WIDE), jnp.float32),
        pltpu.VMEM((_TR, _WIDE), jnp.float32),
        pltpu.VMEM_SHARED((_RA, _WIDE), jnp.float32),
        pltpu.SemaphoreType.DMA,
    ],
)(_pool_sc_body)


def _pool(emb1k, c0, c1):
    embw = jnp.concatenate(
        [emb1k, jnp.ones((NC, 1), jnp.float32),
         jnp.zeros((NC, _WIDE - EMB - 1), jnp.float32)], axis=1)
    pad = _NW * _PW - N
    c0p = jnp.concatenate([c0, jnp.zeros((pad,), jnp.int32)]).reshape(_NW, _CH, _CW)
    c1p = jnp.concatenate([c1, jnp.full((pad,), NC, jnp.int32)]).reshape(_NW, _CH, _CW)
    return _pool_sc_call(embw, c0p, c1p)


def _topk_iter(neg, k):
    iota = jax.lax.broadcasted_iota(jnp.int32, neg.shape, 1).astype(jnp.float32)
    vals, idxs = [], []
    cur = neg
    for _ in range(k):
        m = jnp.max(cur, axis=1, keepdims=True)
        idx = jnp.min(jnp.where(cur == m, iota, 2.0e9), axis=1, keepdims=True)
        vals.append(m)
        idxs.append(idx)
        cur = jnp.where(iota == idx, _NEG_BIG, cur)
    return (jnp.concatenate(vals, axis=1),
            jnp.concatenate(idxs, axis=1).astype(jnp.int32))


def _neg_sqdist(a, b):
    asq = jnp.sum(a * a, axis=1, keepdims=True)
    bsq_row = jax.lax.dot_general(jnp.ones((1, b.shape[1]), jnp.float32), b * b,
                                  (((1,), (1,)), ((), ())),
                                  preferred_element_type=jnp.float32,
                                  precision=jax.lax.Precision.HIGHEST)
    ab = jax.lax.dot_general(a, b, (((1,), (1,)), ((), ())),
                             preferred_element_type=jnp.float32)
    d2 = (asq - 2.0 * ab) + bsq_row
    return -d2


def _combine_partials(acc_ref):
    a = acc_ref[...]
    sums = a[0:NC, 0:EMB] + a[_RA:_RA + NC, 0:EMB]
    cnt = a[0:NC, EMB:EMB + 1] + a[_RA:_RA + NC, EMB:EMB + 1]
    return sums * (1.0 / jnp.clip(cnt, 1.0, None))


def _bip_body(emb_ref, acc_ref, bg_ref, bb_ref,
              idx_ref, logit_ref, w_ref):
    emb = emb_ref[...]
    semb = _combine_partials(acc_ref)
    neg = _neg_sqdist(emb, semb)
    vals, idxs = _topk_iter(neg, BK)
    bg = bg_ref[0, 0]
    bb = bb_ref[0, 0]
    logits = bg * vals + bb
    bw = jnp.exp(logits)
    den = jnp.sum(bw, axis=1, keepdims=True)
    idx_ref[...] = idxs
    logit_ref[...] = logits
    w_ref[...] = bw / (1e-12 + den)


def _bip(emb, acc, bg, bb):
    return pl.pallas_call(
        _bip_body,
        grid=(_NBLK,),
        in_specs=[
            pl.BlockSpec((_ROWS, EMB), lambda i: (i, 0)),
            pl.BlockSpec((2 * _RA, _WIDE), lambda i: (0, 0)),
            pl.BlockSpec((1, 1), lambda i: (0, 0)),
            pl.BlockSpec((1, 1), lambda i: (0, 0)),
        ],
        out_specs=(
            pl.BlockSpec((_ROWS, BK), lambda i: (i, 0)),
            pl.BlockSpec((_ROWS, BK), lambda i: (i, 0)),
            pl.BlockSpec((_ROWS, BK), lambda i: (i, 0)),
        ),
        out_shape=(
            jax.ShapeDtypeStruct((N, BK), jnp.int32),
            jax.ShapeDtypeStruct((N, BK), jnp.float32),
            jax.ShapeDtypeStruct((N, BK), jnp.float32),
        ),
    )(emb, acc, bg.reshape(1, 1), bb.reshape(1, 1))


def _snn_body(acc_ref, sg_ref, sb_ref,
              semb_ref, idx_ref, w1_ref, w2_ref):
    semb = _combine_partials(acc_ref)
    semb_ref[...] = semb
    neg = _neg_sqdist(semb, semb)
    vals, idxs = _topk_iter(neg, SK)
    sg = sg_ref[0, 0]
    sb = sb_ref[0, 0]
    sw = jax.nn.sigmoid(sg * vals + sb)
    iota_r = jax.lax.broadcasted_iota(jnp.int32, (NC, NC), 0)
    iota_c = jax.lax.broadcasted_iota(jnp.int32, (NC, NC), 1)
    diag = iota_r == iota_c
    out_col = jnp.sum(sw, axis=1, keepdims=True)
    s_acc = jnp.zeros((NC, NC), jnp.float32)
    for k in range(SK):
        s_acc += jnp.where(idxs[:, k:k + 1] == iota_c, sw[:, k:k + 1], 0.0)
    in_row = jnp.sum(s_acc, axis=0, keepdims=True)
    out_row = jnp.sum(jnp.where(diag, out_col, 0.0), axis=0, keepdims=True)
    sden_row = out_row + in_row
    sden_col = jnp.sum(jnp.where(diag, sden_row, 0.0), axis=1, keepdims=True)
    w1_ref[...] = sw / (1e-12 + sden_col)
    cols = []
    for k in range(SK):
        picked = jnp.where(idxs[:, k:k + 1] == iota_c, sden_row, 0.0)
        sden_at = jnp.sum(picked, axis=1, keepdims=True)
        cols.append(sw[:, k:k + 1] / (1e-12 + sden_at))
    w2_ref[...] = jnp.concatenate(cols, axis=1)
    idx_ref[...] = idxs


def _snn(acc, sg, sb):
    full = lambda shape: pl.BlockSpec(shape, lambda: (0,) * len(shape))
    return pl.pallas_call(
        _snn_body,
        in_specs=[
            full((2 * _RA, _WIDE)),
            full((1, 1)),
            full((1, 1)),
        ],
        out_specs=(
            full((NC, EMB)),
            full((NC, SK)),
            full((NC, SK)),
            full((NC, SK)),
        ),
        out_shape=(
            jax.ShapeDtypeStruct((NC, EMB), jnp.float32),
            jax.ShapeDtypeStruct((NC, SK), jnp.int32),
            jax.ShapeDtypeStruct((NC, SK), jnp.float32),
            jax.ShapeDtypeStruct((NC, SK), jnp.float32),
        ),
    )(acc, sg.reshape(1, 1), sb.reshape(1, 1))


def kernel(nodes, graph, cluster, W1, b1, W2, b2, W3, b3,
           b_gamma, b_beta, s_gamma, s_beta):
    emb = _mlp(nodes, W1, b1, W2, b2, W3, b3)
    acc = _pool(emb[:NC], cluster[0], cluster[1])
    bnn, blogits5, bw5 = _bip(emb, acc, b_gamma, b_beta)
    semb, snn, sw1, sw2 = _snn(acc, s_gamma, s_beta)

    bsrc = jnp.repeat(jnp.arange(N, dtype=jnp.int32), BK)
    bgraph = jnp.stack([bsrc, bnn.reshape(-1)], axis=0)
    bweights = bw5.reshape(-1, 1)
    blogits = blogits5.reshape(-1)

    s0 = jnp.repeat(jnp.arange(NC, dtype=jnp.int32), SK)
    s1 = snn.reshape(-1)
    sgraph = jnp.stack([jnp.concatenate([s0, s1]),
                        jnp.concatenate([s1, s0])], axis=0)
    sweights = jnp.concatenate([sw1.reshape(-1), sw2.reshape(-1)])[:, None]

    mask = jnp.ones((E,), dtype=bool)
    return (emb, semb, bgraph, bweights, sgraph, sweights, blogits, mask)
